# Initial kernel scaffold; baseline (speedup 1.0000x reference)
import jax
import jax.numpy as jnp
from jax.experimental import pallas as pl


def kernel(x, edge_index, W_lin, b_lin, enc1_W1, enc1_b1, enc1_W2, enc1_b2, enc2_W1, enc2_b1, enc2_W2, enc2_b2, dec1_W1, dec1_b1, dec1_W2, dec1_b2, dec2_W1, dec2_b1, dec2_W2, dec2_b2, Ws, bs):
    raise NotImplementedError("write your pallas kernel here")


# speedup vs baseline: 2.5980x; 2.5980x over previous
import functools

import jax
import jax.numpy as jnp
from jax import lax
from jax.experimental import pallas as pl
from jax.experimental.pallas import tpu as pltpu
from jax.experimental.pallas import tpu_sc as plsc

_N = 10000
_E = 320000
_IN_DIM = 128
_HID = 64

_NC = 2
_NS = 16
_NW = _NC * _NS
_CHUNK = 128
_CPW = 80
_EPW = _CPW * _CHUNK
_E_PAD = _NW * _EPW
_N_ACC = 10240
_ZROWS = _N_ACC // _NS
_RPS = _N // _NS

_MESH = plsc.VectorSubcoreMesh(core_axis_name="c", subcore_axis_name="s")


@functools.partial(
    pl.kernel,
    out_type=jax.ShapeDtypeStruct((_NC, _N_ACC, _HID), jnp.float32),
    mesh=_MESH,
    scratch_types=[
        pltpu.VMEM((_CPW, _CHUNK), jnp.int32),
        pltpu.VMEM((_CPW, _CHUNK), jnp.int32),
        pltpu.VMEM((_CHUNK, _HID), jnp.float32),
        pltpu.VMEM((_ZROWS, _HID), jnp.float32),
        pltpu.VMEM_SHARED((_N_ACC, _HID), jnp.float32),
        pltpu.SemaphoreType.DMA,
    ],
    compiler_params=pltpu.CompilerParams(use_tc_tiling_on_sc=False),
)
def _sc_scatter_add(h_hbm, src_hbm, dst_hbm, z_hbm, out_hbm,
                    src_v, dst_v, rows_v, stage_v, acc, sem):
    c = lax.axis_index("c")
    s = lax.axis_index("s")
    wid = s * _NC + c
    pltpu.sync_copy(z_hbm, stage_v)
    pltpu.sync_copy(stage_v, acc.at[pl.ds(s * _ZROWS, _ZROWS)])
    pltpu.sync_copy(src_hbm.at[wid], src_v)
    pltpu.sync_copy(dst_hbm.at[wid], dst_v)
    plsc.subcore_barrier()

    def body(j, carry):
        pltpu.async_copy(h_hbm.at[src_v.at[j]], rows_v, sem).wait()
        pltpu.sync_copy(rows_v, acc.at[dst_v.at[j]], add=True)
        return carry

    lax.fori_loop(0, _CPW, body, 0)
    plsc.subcore_barrier()
    pltpu.sync_copy(acc.at[pl.ds(s * _ZROWS, _ZROWS)], stage_v)
    pltpu.sync_copy(stage_v, out_hbm.at[c, pl.ds(s * _ZROWS, _ZROWS)])


@functools.partial(
    pl.kernel,
    out_type=jax.ShapeDtypeStruct((_NW, _CPW, _CHUNK), jnp.float32),
    mesh=_MESH,
    scratch_types=[
        pltpu.VMEM((_CPW, _CHUNK), jnp.int32),
        pltpu.VMEM((_CPW, _CHUNK), jnp.int32),
        pltpu.VMEM((_CHUNK, _HID), jnp.float32),
        pltpu.VMEM((_CHUNK, _HID), jnp.float32),
        pltpu.VMEM((_CPW, _CHUNK), jnp.float32),
        pltpu.SemaphoreType.DMA,
        pltpu.SemaphoreType.DMA,
    ],
    compiler_params=pltpu.CompilerParams(use_tc_tiling_on_sc=False,
                                         needs_layout_passes=False),
)
def _sc_edge_dot(hs_hbm, src_hbm, dst_hbm, out_hbm,
                 src_v, dst_v, a_v, b_v, o_v, sem_a, sem_b):
    c = lax.axis_index("c")
    s = lax.axis_index("s")
    wid = s * _NC + c
    pltpu.sync_copy(src_hbm.at[wid], src_v)
    pltpu.sync_copy(dst_hbm.at[wid], dst_v)

    def chunk_body(j, carry):
        ca = pltpu.async_copy(hs_hbm.at[src_v.at[j]], a_v, sem_a)
        cb = pltpu.async_copy(hs_hbm.at[dst_v.at[j]], b_v, sem_b)
        ca.wait()
        cb.wait()

        def grp_body(g, gcarry):
            e16 = lax.iota(jnp.int32, 16) + g * 16
            accs = [jnp.zeros((16,), jnp.float32) for _ in range(4)]
            for d in range(_HID):
                dv = jnp.full((16,), d, jnp.int32)
                av = plsc.load_gather(a_v, [e16, dv])
                bv = plsc.load_gather(b_v, [e16, dv])
                accs[d % 4] = accs[d % 4] + av * bv
            o_v[j, pl.ds(g * 16, 16)] = (accs[0] + accs[1]) + (accs[2] + accs[3])
            return gcarry

        lax.fori_loop(0, _CHUNK // 16, grp_body, 0)
        return carry

    lax.fori_loop(0, _CPW, chunk_body, 0)
    pltpu.sync_copy(o_v, out_hbm.at[wid])


_BLK = 1000


def _tc_linear(x, w, b, relu):
    n, k = x.shape
    m = w.shape[1]

    def body(x_ref, w_ref, b_ref, o_ref):
        y = lax.dot_general(x_ref[...], w_ref[...], (((1,), (0,)), ((), ())),
                            preferred_element_type=jnp.float32) + b_ref[...]
        o_ref[...] = jnp.maximum(y, 0.0) if relu else y

    return pl.pallas_call(
        body,
        grid=(n // _BLK,),
        in_specs=[
            pl.BlockSpec((_BLK, k), lambda i: (i, 0)),
            pl.BlockSpec((k, m), lambda i: (0, 0)),
            pl.BlockSpec((1, m), lambda i: (0, 0)),
        ],
        out_specs=pl.BlockSpec((_BLK, m), lambda i: (i, 0)),
        out_shape=jax.ShapeDtypeStruct((n, m), jnp.float32),
    )(x, w, b.reshape(1, m))


def _tc_gin_mlp(h, agg, w1, b1, w2, b2, relu_out):
    n = h.shape[0]
    m = w2.shape[1]

    def body(h_ref, a_ref, w1_ref, b1_ref, w2_ref, b2_ref, o_ref):
        z = h_ref[...] + a_ref[0] + a_ref[1]
        t = lax.dot_general(z, w1_ref[...], (((1,), (0,)), ((), ())),
                            preferred_element_type=jnp.float32) + b1_ref[...]
        t = jnp.maximum(t, 0.0)
        y = lax.dot_general(t, w2_ref[...], (((1,), (0,)), ((), ())),
                            preferred_element_type=jnp.float32) + b2_ref[...]
        o_ref[...] = jnp.maximum(y, 0.0) if relu_out else y

    return pl.pallas_call(
        body,
        grid=(n // _BLK,),
        in_specs=[
            pl.BlockSpec((_BLK, _HID), lambda i: (i, 0)),
            pl.BlockSpec((_NC, _BLK, _HID), lambda i: (0, i, 0)),
            pl.BlockSpec((_HID, _HID), lambda i: (0, 0)),
            pl.BlockSpec((1, _HID), lambda i: (0, 0)),
            pl.BlockSpec((_HID, m), lambda i: (0, 0)),
            pl.BlockSpec((1, m), lambda i: (0, 0)),
        ],
        out_specs=pl.BlockSpec((_BLK, m), lambda i: (i, 0)),
        out_shape=jax.ShapeDtypeStruct((n, m), jnp.float32),
    )(h, agg, w1, b1.reshape(1, _HID), w2, b2.reshape(1, m))


def kernel(x, edge_index, W_lin, b_lin, enc1_W1, enc1_b1, enc1_W2, enc1_b2,
           enc2_W1, enc2_b1, enc2_W2, enc2_b2, dec1_W1, dec1_b1, dec1_W2,
           dec1_b2, dec2_W1, dec2_b1, dec2_W2, dec2_b2, Ws, bs):
    src = edge_index[0]
    dst = edge_index[1]
    pad = _E_PAD - _E
    srcp = jnp.concatenate(
        [src, jnp.zeros((pad,), jnp.int32)]).reshape(_NW, _CPW, _CHUNK)
    dst_sc = jnp.concatenate(
        [dst, jnp.full((pad,), _N, jnp.int32)]).reshape(_NW, _CPW, _CHUNK)
    dst_g = jnp.concatenate(
        [dst, jnp.zeros((pad,), jnp.int32)]).reshape(_NW, _CPW, _CHUNK)
    zeros_blk = jnp.zeros((_ZROWS, _HID), jnp.float32)

    def gin(h, w1, b1, w2, b2, relu_out):
        agg = _sc_scatter_add(h, srcp, dst_sc, zeros_blk)
        return _tc_gin_mlp(h, agg, w1, b1, w2, b2, relu_out)

    h0 = _tc_linear(x, W_lin, b_lin, relu=False)
    h1 = gin(h0, enc1_W1, enc1_b1, enc1_W2, enc1_b2, True)
    emb = gin(h1, enc2_W1, enc2_b1, enc2_W2, enc2_b2, False)
    a = gin(emb, dec1_W1, dec1_b1, dec1_W2, dec1_b2, True)
    x_ = gin(a, dec2_W1, dec2_b1, dec2_W2, dec2_b2, False)

    hs = _tc_linear(emb, Ws, bs, relu=True)
    s_pad = _sc_edge_dot(hs, srcp, dst_g)
    s_ = s_pad.reshape(-1)[:_E]
    return (x_, s_)
